# Initial kernel scaffold; baseline (speedup 1.0000x reference)
import jax
import jax.numpy as jnp
from jax.experimental import pallas as pl


def kernel(x, edge_index, edge_attr, node_mask, edge_mask, W1, b1, W2, b2, W3, b3, W_root, bias):
    raise NotImplementedError("write your pallas kernel here")


# speedup vs baseline: 3.1092x; 3.1092x over previous
import functools

import jax
import jax.numpy as jnp
from jax import lax
from jax.experimental import pallas as pl
from jax.experimental.pallas import tpu as pltpu
from jax.experimental.pallas import tpu_sc as plsc

N = 10000
E = 320000
F = 128
OUT = 3

_NC = 2
_NS = 16
_NW = _NC * _NS
_CH = 128
_NCHUNK = E // _CH
_JMAX = (_NCHUNK + _NW - 1) // _NW

@functools.lru_cache(maxsize=None)
def _sc_kernels():
    mesh = plsc.VectorSubcoreMesh(
        core_axis_name="c", subcore_axis_name="s", num_cores=_NC,
        num_subcores=_NS)

    @functools.partial(
        pl.kernel,
        mesh=mesh,
        out_type=jax.ShapeDtypeStruct((E, F), jnp.float32),
        scratch_types=[
            pltpu.VMEM((_CH,), jnp.int32),
            pltpu.VMEM((_CH, F), jnp.float32),
            pltpu.SemaphoreType.DMA,
        ],
    )
    def sc_gather(x_hbm, src_hbm, out_hbm, idx_v, rows_v, sem):
        wid = lax.axis_index("s") * _NC + lax.axis_index("c")

        def body(j, carry):
            chunk = wid + _NW * j

            @pl.when(chunk < _NCHUNK)
            def _():
                base = chunk * _CH
                pltpu.sync_copy(src_hbm.at[pl.ds(base, _CH)], idx_v)
                pltpu.async_copy(x_hbm.at[idx_v], rows_v, sem).wait()
                pltpu.sync_copy(rows_v, out_hbm.at[pl.ds(base, _CH)])

            return carry

        lax.fori_loop(0, _JMAX, body, 0)

    @functools.partial(
        pl.kernel,
        mesh=mesh,
        out_type=jax.ShapeDtypeStruct((_NC, N, 128), jnp.float32),
        scratch_types=[
            pltpu.VMEM((_CH,), jnp.int32),
            pltpu.VMEM((_CH, 128), jnp.float32),
            pltpu.VMEM_SHARED((N, 128), jnp.float32),
        ],
    )
    def sc_scatter(msg_hbm, dst_hbm, zero_hbm, out_hbm, idx_v, msg_v, acc):
        cid = lax.axis_index("c")
        sid = lax.axis_index("s")
        wid = sid * _NC + cid

        @pl.when(sid == 0)
        def _():
            pltpu.sync_copy(zero_hbm, acc)

        plsc.subcore_barrier()

        def body(j, carry):
            chunk = wid + _NW * j

            @pl.when(chunk < _NCHUNK)
            def _():
                base = chunk * _CH
                pltpu.sync_copy(dst_hbm.at[pl.ds(base, _CH)], idx_v)
                pltpu.sync_copy(msg_hbm.at[pl.ds(base, _CH)], msg_v)
                pltpu.sync_copy(msg_v, acc.at[idx_v], add=True)

            return carry

        lax.fori_loop(0, _JMAX, body, 0)
        plsc.subcore_barrier()

        @pl.when(sid == 0)
        def _():
            pltpu.sync_copy(acc, out_hbm.at[cid])

    return sc_gather, sc_scatter


def _elu(v):
    return jnp.where(v > 0, v, jnp.exp(jnp.minimum(v, 0.0)) - 1.0)


def _msg_body(ea_ref, xg_ref, w1t_ref, b1_ref, w2t_ref, b2_ref, a_ref,
              bb_ref, msg_ref):
    h1 = jnp.dot(ea_ref[...], w1t_ref[...],
                 preferred_element_type=jnp.float32) + b1_ref[...]
    h1 = _elu(h1)
    h2 = jnp.dot(h1, w2t_ref[...],
                 preferred_element_type=jnp.float32) + b2_ref[...]
    h2 = _elu(h2)
    xg = xg_ref[...]
    y = jnp.dot(xg, a_ref[...], preferred_element_type=jnp.float32)
    xb = jnp.dot(xg, bb_ref[...], preferred_element_type=jnp.float32)
    m0 = jnp.sum(h2 * y[:, 0 * F:1 * F], axis=1, keepdims=True)
    m1 = jnp.sum(h2 * y[:, 1 * F:2 * F], axis=1, keepdims=True)
    m2 = jnp.sum(h2 * y[:, 2 * F:3 * F], axis=1, keepdims=True)
    ones = jnp.ones_like(m0)
    pad = jnp.zeros((m0.shape[0], 124), jnp.float32)
    msg_ref[...] = jnp.concatenate(
        [jnp.concatenate([m0, m1, m2, ones], axis=1) + xb, pad], axis=1)


def _msg_call(ea, xg, w1t, b1r, w2t, b2r, a, bb):
    bq = 512
    grid = (E // bq,)
    return pl.pallas_call(
        _msg_body,
        grid=grid,
        in_specs=[
            pl.BlockSpec((bq, F), lambda i: (i, 0)),
            pl.BlockSpec((bq, F), lambda i: (i, 0)),
            pl.BlockSpec((F, 256), lambda i: (0, 0)),
            pl.BlockSpec((1, 256), lambda i: (0, 0)),
            pl.BlockSpec((256, F), lambda i: (0, 0)),
            pl.BlockSpec((1, F), lambda i: (0, 0)),
            pl.BlockSpec((F, OUT * F), lambda i: (0, 0)),
            pl.BlockSpec((F, 4), lambda i: (0, 0)),
        ],
        out_specs=pl.BlockSpec((bq, 128), lambda i: (i, 0)),
        out_shape=jax.ShapeDtypeStruct((E, 128), jnp.float32),
        compiler_params=pltpu.CompilerParams(
            dimension_semantics=("arbitrary",)),
    )(ea, xg, w1t, b1r, w2t, b2r, a, bb)


def _fin_body(x_ref, p_ref, wrt_ref, bias_ref, out_ref):
    s = (p_ref[0] + p_ref[1])[:, :4]
    cnt = s[:, 3:4]
    aggr = s / jnp.clip(cnt, 1.0, None)
    logits = jnp.dot(x_ref[...], wrt_ref[...],
                     preferred_element_type=jnp.float32) + aggr + bias_ref[...]
    m = jnp.max(logits, axis=0, keepdims=True)
    e = jnp.exp(logits - m)
    out_ref[...] = e / jnp.sum(e, axis=0, keepdims=True)


def _fin_call(xx, parts, wrtp, biasp):
    return pl.pallas_call(
        _fin_body,
        out_shape=jax.ShapeDtypeStruct((N, 4), jnp.float32),
    )(xx, parts, wrtp, biasp)


def kernel(x, edge_index, edge_attr, node_mask, edge_mask,
           W1, b1, W2, b2, W3, b3, W_root, bias):
    xx = jnp.where(node_mask[:, None], x, 0.0)
    edges = jnp.where(edge_mask[None, :], edge_index, 0)
    ea = jnp.where(edge_mask[:, None], edge_attr, 0.0)
    src = edges[0]
    dst = edges[1]

    w1t = W1.T
    b1r = b1.reshape(1, 256)
    w2t = W2.T
    b2r = b2.reshape(1, F)
    a = W3.reshape(F, OUT, F).reshape(F, OUT * F)
    bb = jnp.concatenate(
        [b3.reshape(F, OUT), jnp.zeros((F, 1), jnp.float32)], axis=1)
    wrtp = jnp.concatenate(
        [W_root.T, jnp.zeros((F, 1), jnp.float32)], axis=1)
    biasp = jnp.concatenate(
        [bias, jnp.zeros((1,), jnp.float32)]).reshape(1, 4)

    sc_gather, sc_scatter = _sc_kernels()
    xg = sc_gather(xx, src)
    msg = _msg_call(ea, xg, w1t, b1r, w2t, b2r, a, bb)
    parts = sc_scatter(msg, dst, jnp.zeros((N, 128), jnp.float32))
    out4 = _fin_call(xx, parts, wrtp, biasp)
    return out4[:, :OUT]
